# baseline (speedup 1.0000x reference)
import functools

import jax
import jax.numpy as jnp
from jax import lax
from jax.experimental import pallas as pl
from jax.experimental.pallas import tpu as pltpu
from jax.experimental.pallas import tpu_sc as plsc

N = 10000
E = 320000

NUM_CORES = 2
NUM_SUBCORES = 16
NUM_WORKERS = NUM_CORES * NUM_SUBCORES
K_EDGES = 128
EDGES_PER_WORKER = 10240
E_PAD = EDGES_PER_WORKER * NUM_WORKERS
CHUNKS = EDGES_PER_WORKER // K_EDGES
ROWS_PER_TILE = N // NUM_SUBCORES
ZROWS = 125


def _make_spmm(do: int):
    mesh = plsc.VectorSubcoreMesh(
        core_axis_name="c", subcore_axis_name="s",
        num_cores=NUM_CORES, num_subcores=NUM_SUBCORES)

    @functools.partial(
        pl.kernel,
        out_type=jax.ShapeDtypeStruct((NUM_CORES, N, do), jnp.float32),
        mesh=mesh,
        compiler_params=pltpu.CompilerParams(
            needs_layout_passes=False, use_tc_tiling_on_sc=False),
        scratch_types=[
            pltpu.VMEM((3, K_EDGES), jnp.int32),
            pltpu.VMEM((3, K_EDGES), jnp.int32),
            pltpu.VMEM((K_EDGES, do), jnp.float32),
            pltpu.VMEM((K_EDGES, do), jnp.float32),
            pltpu.VMEM((ZROWS, do), jnp.float32),
            pltpu.VMEM_SHARED((N, do), jnp.float32),
            pltpu.SemaphoreType.DMA,
            pltpu.SemaphoreType.DMA,
            pltpu.SemaphoreType.DMA,
            pltpu.SemaphoreType.DMA,
        ],
    )
    def spmm(support_hbm, edata_hbm, out_hbm,
             eb0, eb1, rows0, rows1, zbuf, acc, se0, se1, sg0, sg1):
        cid = lax.axis_index("c")
        sid = lax.axis_index("s")
        wid = sid * NUM_CORES + cid
        row0 = sid * ROWS_PER_TILE

        zero16 = jnp.zeros((16,), jnp.float32)

        def zfill(i, carry):
            for j in range(do // 16):
                zbuf[i, pl.ds(j * 16, 16)] = zero16
            return carry

        lax.fori_loop(0, ZROWS, zfill, 0)
        for z in range(ROWS_PER_TILE // ZROWS):
            pltpu.sync_copy(zbuf, acc.at[pl.ds(row0 + z * ZROWS, ZROWS)])
        plsc.subcore_barrier()

        eb = (eb0, eb1)
        rows = (rows0, rows1)
        se = (se0, se1)
        sg = (sg0, sg1)
        iota16 = lax.iota(jnp.int32, 16)
        row_idx = [iota16 + 16 * g for g in range(K_EDGES // 16)]

        def process(j, p):
            q = 1 - p
            pltpu.make_async_copy(edata_hbm.at[wid, j], eb[q], se[q]).wait()
            pltpu.async_copy(support_hbm.at[eb[q].at[0]], rows[q], sg[q])
            pltpu.make_async_copy(support_hbm.at[eb[p].at[0]], rows[p],
                                  sg[p]).wait()
            adj16 = [plsc.bitcast(eb[p][2, pl.ds(16 * g, 16)], jnp.float32)
                     for g in range(K_EDGES // 16)]

            def scale_col(c, c2):
                col = jnp.full((16,), 0, jnp.int32) + c
                for g in range(K_EDGES // 16):
                    v = plsc.load_gather(rows[p], [row_idx[g], col])
                    plsc.store_scatter(rows[p], [row_idx[g], col],
                                       v * adj16[g])
                return c2

            lax.fori_loop(0, do, scale_col, 0)
            pltpu.sync_copy(rows[p], acc.at[eb[p].at[1]], add=True)
            pltpu.async_copy(edata_hbm.at[wid, j + 2], eb[p], se[p])

        pltpu.sync_copy(edata_hbm.at[wid, 0], eb0)
        pltpu.async_copy(edata_hbm.at[wid, 1], eb1, se1)
        pltpu.async_copy(support_hbm.at[eb0.at[0]], rows0, sg0)

        def pair(t, carry):
            process(2 * t, 0)
            process(2 * t + 1, 1)
            return carry

        lax.fori_loop(0, CHUNKS // 2, pair, 0)
        pltpu.make_async_copy(edata_hbm.at[wid, 0], eb1, se1).wait()
        pltpu.make_async_copy(support_hbm.at[eb0.at[0]], rows0, sg0).wait()

        plsc.subcore_barrier()
        pltpu.sync_copy(acc.at[pl.ds(row0, ROWS_PER_TILE)],
                        out_hbm.at[cid, pl.ds(row0, ROWS_PER_TILE)])

    return spmm


_spmm = {d: _make_spmm(d) for d in (16, 32, 64, 128)}


def _mm0_body(x_ref, w_ref, o_ref):
    o_ref[...] = jnp.dot(x_ref[...], w_ref[...],
                         preferred_element_type=jnp.float32)


def _tc_mm0(x, w):
    return pl.pallas_call(
        _mm0_body,
        out_shape=jax.ShapeDtypeStruct((N, w.shape[1]), jnp.float32),
    )(x, w)


def _fused_body(p0_ref, p1_ref, b_ref, g_ref, beta_ref, w_ref, o_ref):
    s = p0_ref[...] + p1_ref[...] + b_ref[...]
    mu = jnp.mean(s, axis=0, keepdims=True)
    xc = s - mu
    var = jnp.mean(xc * xc, axis=0, keepdims=True)
    h = xc * lax.rsqrt(var + 1e-5) * g_ref[...] + beta_ref[...]
    h = jnp.maximum(h, 0.0)
    o_ref[...] = jnp.dot(h, w_ref[...], preferred_element_type=jnp.float32)


def _tc_fused(p0, p1, b, g, beta, w):
    return pl.pallas_call(
        _fused_body,
        out_shape=jax.ShapeDtypeStruct((N, w.shape[1]), jnp.float32),
    )(p0, p1, b.reshape(1, -1), g.reshape(1, -1), beta.reshape(1, -1), w)


def _final_body(p0_ref, p1_ref, b_ref, zm_ref, zs_ref):
    q = p0_ref[...] + p1_ref[...] + b_ref[...]
    zm_ref[...] = q[:, :32]
    zs_ref[...] = q[:, 32:]


def _tc_final(p0, p1, b67):
    return pl.pallas_call(
        _final_body,
        out_shape=(jax.ShapeDtypeStruct((N, 32), jnp.float32),
                   jax.ShapeDtypeStruct((N, 32), jnp.float32)),
    )(p0, p1, b67.reshape(1, -1))


def kernel(x, edge_index, adj_values, W1, b1, W2, b2, W3, b3, W4, b4,
           W5, b5, W6, b6, W7, b7, g1, beta1, g2, beta2, g3, beta3,
           g4, beta4, g5, beta5):
    pad = E_PAD - E
    src = jnp.concatenate([edge_index[0], jnp.zeros((pad,), jnp.int32)])
    dst = jnp.concatenate([edge_index[1], jnp.zeros((pad,), jnp.int32)])
    adj = jnp.concatenate([adj_values, jnp.zeros((pad,), jnp.float32)])
    edata = jnp.stack(
        [src.reshape(NUM_WORKERS, CHUNKS, K_EDGES),
         dst.reshape(NUM_WORKERS, CHUNKS, K_EDGES),
         lax.bitcast_convert_type(adj, jnp.int32).reshape(
             NUM_WORKERS, CHUNKS, K_EDGES)], axis=2)
    edata = jnp.pad(edata, ((0, 0), (0, 2), (0, 0), (0, 0)))

    def spmm(support):
        p = _spmm[support.shape[1]](support, edata)
        return p[0], p[1]

    sup = _tc_mm0(x, W1)
    p0, p1 = spmm(sup)
    sup = _tc_fused(p0, p1, b1, g1, beta1, W2)
    p0, p1 = spmm(sup)
    sup = _tc_fused(p0, p1, b2, g2, beta2, W3)
    p0, p1 = spmm(sup)
    sup = _tc_fused(p0, p1, b3, g3, beta3, W4)
    p0, p1 = spmm(sup)
    sup = _tc_fused(p0, p1, b4, g4, beta4, W5)
    p0, p1 = spmm(sup)
    W67 = jnp.concatenate([W6, W7], axis=1)
    sup = _tc_fused(p0, p1, b5, g5, beta5, W67)
    p0, p1 = spmm(sup)
    b67 = jnp.concatenate([b6, b7])
    z_mean, z_std = _tc_final(p0, p1, b67)
    return (z_mean, z_std)


# speedup vs baseline: 2.4832x; 2.4832x over previous
import functools

import jax
import jax.numpy as jnp
from jax import lax
from jax.experimental import pallas as pl
from jax.experimental.pallas import tpu as pltpu
from jax.experimental.pallas import tpu_sc as plsc

N = 10000
E = 320000

NUM_CORES = 2
NUM_SUBCORES = 16
NUM_WORKERS = NUM_CORES * NUM_SUBCORES
K_EDGES = 128
EDGES_PER_WORKER = 10240
E_PAD = EDGES_PER_WORKER * NUM_WORKERS
CHUNKS = EDGES_PER_WORKER // K_EDGES
ROWS_PER_TILE = N // NUM_SUBCORES
ZROWS = 125


def _make_spmm(do: int):
    mesh = plsc.VectorSubcoreMesh(
        core_axis_name="c", subcore_axis_name="s",
        num_cores=NUM_CORES, num_subcores=NUM_SUBCORES)

    @functools.partial(
        pl.kernel,
        out_type=jax.ShapeDtypeStruct((NUM_CORES, N, do), jnp.float32),
        mesh=mesh,
        compiler_params=pltpu.CompilerParams(
            needs_layout_passes=False, use_tc_tiling_on_sc=False),
        scratch_types=[
            pltpu.VMEM((3, K_EDGES), jnp.int32),
            pltpu.VMEM((3, K_EDGES), jnp.int32),
            pltpu.VMEM((K_EDGES, do), jnp.float32),
            pltpu.VMEM((K_EDGES, do), jnp.float32),
            pltpu.VMEM((ZROWS, do), jnp.float32),
            pltpu.VMEM_SHARED((N, do), jnp.float32),
            pltpu.SemaphoreType.DMA,
            pltpu.SemaphoreType.DMA,
            pltpu.SemaphoreType.DMA,
            pltpu.SemaphoreType.DMA,
        ],
    )
    def spmm(support_hbm, edata_hbm, out_hbm,
             eb0, eb1, rows0, rows1, zbuf, acc, se0, se1, sg0, sg1):
        cid = lax.axis_index("c")
        sid = lax.axis_index("s")
        wid = sid * NUM_CORES + cid
        row0 = sid * ROWS_PER_TILE

        zero16 = jnp.zeros((16,), jnp.float32)

        def zfill(i, carry):
            for j in range(do // 16):
                zbuf[i, pl.ds(j * 16, 16)] = zero16
            return carry

        lax.fori_loop(0, ZROWS, zfill, 0)
        for z in range(ROWS_PER_TILE // ZROWS):
            pltpu.sync_copy(zbuf, acc.at[pl.ds(row0 + z * ZROWS, ZROWS)])
        plsc.subcore_barrier()

        eb = (eb0, eb1)
        rows = (rows0, rows1)
        se = (se0, se1)
        sg = (sg0, sg1)
        zero_idx = jnp.full((16,), 0, jnp.int32)

        def process(j, p):
            q = 1 - p
            pltpu.make_async_copy(edata_hbm.at[wid, j], eb[q], se[q]).wait()
            pltpu.async_copy(support_hbm.at[eb[q].at[0]], rows[q], sg[q])
            pltpu.make_async_copy(support_hbm.at[eb[p].at[0]], rows[p],
                                  sg[p]).wait()
            def scale(e, c2):
                a16i = plsc.load_gather(eb[p].at[2], [zero_idx + e])
                a16 = plsc.bitcast(a16i, jnp.float32)
                for jj in range(do // 16):
                    sl = rows[p][e, pl.ds(jj * 16, 16)]
                    rows[p][e, pl.ds(jj * 16, 16)] = sl * a16
                return c2

            lax.fori_loop(0, K_EDGES, scale, 0)
            pltpu.sync_copy(rows[p], acc.at[eb[p].at[1]], add=True)
            pltpu.async_copy(edata_hbm.at[wid, j + 2], eb[p], se[p])

        pltpu.sync_copy(edata_hbm.at[wid, 0], eb0)
        pltpu.async_copy(edata_hbm.at[wid, 1], eb1, se1)
        pltpu.async_copy(support_hbm.at[eb0.at[0]], rows0, sg0)

        def pair(t, carry):
            process(2 * t, 0)
            process(2 * t + 1, 1)
            return carry

        lax.fori_loop(0, CHUNKS // 2, pair, 0)
        pltpu.make_async_copy(edata_hbm.at[wid, 0], eb1, se1).wait()
        pltpu.make_async_copy(support_hbm.at[eb0.at[0]], rows0, sg0).wait()

        plsc.subcore_barrier()
        pltpu.sync_copy(acc.at[pl.ds(row0, ROWS_PER_TILE)],
                        out_hbm.at[cid, pl.ds(row0, ROWS_PER_TILE)])

    return spmm


_spmm = {d: _make_spmm(d) for d in (16, 32, 64, 128)}


def _mm0_body(x_ref, w_ref, o_ref):
    o_ref[...] = jnp.dot(x_ref[...], w_ref[...],
                         preferred_element_type=jnp.float32)


def _tc_mm0(x, w):
    return pl.pallas_call(
        _mm0_body,
        out_shape=jax.ShapeDtypeStruct((N, w.shape[1]), jnp.float32),
    )(x, w)


def _fused_body(p0_ref, p1_ref, b_ref, g_ref, beta_ref, w_ref, o_ref):
    s = p0_ref[...] + p1_ref[...] + b_ref[...]
    mu = jnp.mean(s, axis=0, keepdims=True)
    xc = s - mu
    var = jnp.mean(xc * xc, axis=0, keepdims=True)
    h = xc * lax.rsqrt(var + 1e-5) * g_ref[...] + beta_ref[...]
    h = jnp.maximum(h, 0.0)
    o_ref[...] = jnp.dot(h, w_ref[...], preferred_element_type=jnp.float32)


def _tc_fused(p0, p1, b, g, beta, w):
    return pl.pallas_call(
        _fused_body,
        out_shape=jax.ShapeDtypeStruct((N, w.shape[1]), jnp.float32),
    )(p0, p1, b.reshape(1, -1), g.reshape(1, -1), beta.reshape(1, -1), w)


def _final_body(p0_ref, p1_ref, b_ref, zm_ref, zs_ref):
    q = p0_ref[...] + p1_ref[...] + b_ref[...]
    zm_ref[...] = q[:, :32]
    zs_ref[...] = q[:, 32:]


def _tc_final(p0, p1, b67):
    return pl.pallas_call(
        _final_body,
        out_shape=(jax.ShapeDtypeStruct((N, 32), jnp.float32),
                   jax.ShapeDtypeStruct((N, 32), jnp.float32)),
    )(p0, p1, b67.reshape(1, -1))


def kernel(x, edge_index, adj_values, W1, b1, W2, b2, W3, b3, W4, b4,
           W5, b5, W6, b6, W7, b7, g1, beta1, g2, beta2, g3, beta3,
           g4, beta4, g5, beta5):
    pad = E_PAD - E
    src = jnp.concatenate([edge_index[0], jnp.zeros((pad,), jnp.int32)])
    dst = jnp.concatenate([edge_index[1], jnp.zeros((pad,), jnp.int32)])
    adj = jnp.concatenate([adj_values, jnp.zeros((pad,), jnp.float32)])
    edata = jnp.stack(
        [src.reshape(NUM_WORKERS, CHUNKS, K_EDGES),
         dst.reshape(NUM_WORKERS, CHUNKS, K_EDGES),
         lax.bitcast_convert_type(adj, jnp.int32).reshape(
             NUM_WORKERS, CHUNKS, K_EDGES)], axis=2)
    edata = jnp.pad(edata, ((0, 0), (0, 2), (0, 0), (0, 0)))

    def spmm(support):
        p = _spmm[support.shape[1]](support, edata)
        return p[0], p[1]

    sup = _tc_mm0(x, W1)
    p0, p1 = spmm(sup)
    sup = _tc_fused(p0, p1, b1, g1, beta1, W2)
    p0, p1 = spmm(sup)
    sup = _tc_fused(p0, p1, b2, g2, beta2, W3)
    p0, p1 = spmm(sup)
    sup = _tc_fused(p0, p1, b3, g3, beta3, W4)
    p0, p1 = spmm(sup)
    sup = _tc_fused(p0, p1, b4, g4, beta4, W5)
    p0, p1 = spmm(sup)
    W67 = jnp.concatenate([W6, W7], axis=1)
    sup = _tc_fused(p0, p1, b5, g5, beta5, W67)
    p0, p1 = spmm(sup)
    b67 = jnp.concatenate([b6, b7])
    z_mean, z_std = _tc_final(p0, p1, b67)
    return (z_mean, z_std)
